# baseline (speedup 1.0000x reference)
import functools
import numpy as np
import jax
import jax.numpy as jnp
from jax.experimental import pallas as pl
from jax.experimental.pallas import tpu as pltpu

_N = 50000
_D = 128
_H = 4
_DH = _D // _H
_E = 300000
_CHUNK = 1024
_EPAD = 300032
_ROWB = 2000

_HEAD_MASK = np.kron(np.eye(_H, dtype=np.float32),
                     np.ones((_DH, _DH), dtype=np.float32))


def _proj_body(x_ref, w_ref, b_ref, o_ref):
    o_ref[...] = (
        jnp.dot(x_ref[...], w_ref[...], preferred_element_type=jnp.float32)
        + b_ref[...]
    )


def _proj(x, w, b):
    return pl.pallas_call(
        _proj_body,
        grid=(_N // _ROWB,),
        in_specs=[
            pl.BlockSpec((_ROWB, _D), lambda i: (i, 0)),
            pl.BlockSpec((_D, _D), lambda i: (0, 0)),
            pl.BlockSpec((1, _D), lambda i: (0, 0)),
        ],
        out_specs=pl.BlockSpec((_ROWB, _D), lambda i: (i, 0)),
        out_shape=jax.ShapeDtypeStruct((_N, _D), jnp.float32),
        compiler_params=pltpu.CompilerParams(
            dimension_semantics=("parallel",)),
    )(x, w, b.reshape(1, _D))


def _alpha_body(idx_ref, krel_ref, q_ref, mask_ref, exb_ref):
    def body(e, carry):
        s = idx_ref[0, e]
        d = idx_ref[1, e]
        krow = krel_ref[pl.ds(s, 1), :]
        qrow = q_ref[pl.ds(d, 1), :]
        arow = jnp.dot(krow * qrow, mask_ref[...],
                       preferred_element_type=jnp.float32)
        exb_ref[pl.ds(e, 1), :] = jnp.exp(arow)
        return carry
    n = jnp.minimum(_CHUNK, _E - pl.program_id(0) * _CHUNK)
    jax.lax.fori_loop(0, n, body, 0)


def _edge_alpha(edge, krel, q):
    mask = jnp.asarray(_HEAD_MASK)
    return pl.pallas_call(
        _alpha_body,
        grid=(_EPAD // _CHUNK,),
        in_specs=[
            pl.BlockSpec((2, _CHUNK), lambda i: (0, i),
                         memory_space=pltpu.SMEM),
            pl.BlockSpec((_N, _D), lambda i: (0, 0)),
            pl.BlockSpec((_N, _D), lambda i: (0, 0)),
            pl.BlockSpec((_D, _D), lambda i: (0, 0)),
        ],
        out_specs=pl.BlockSpec((_CHUNK, _D), lambda i: (i, 0)),
        out_shape=jax.ShapeDtypeStruct((_EPAD, _D), jnp.float32),
        compiler_params=pltpu.CompilerParams(
            dimension_semantics=("parallel",)),
    )(edge, krel, q, mask)


def _msgscatter_body(idx_ref, exb_ref, vrel_ref, num_ref):
    @pl.when(pl.program_id(0) == 0)
    def _():
        num_ref[...] = jnp.zeros_like(num_ref)

    def body(e, carry):
        s = idx_ref[0, e]
        d = idx_ref[1, e]
        vrow = vrel_ref[pl.ds(s, 1), :]
        exrow = exb_ref[pl.ds(e, 1), :]
        cur = num_ref[pl.ds(d, 1), :]
        num_ref[pl.ds(d, 1), :] = cur + exrow * vrow
        return carry
    n = jnp.minimum(_CHUNK, _E - pl.program_id(0) * _CHUNK)
    jax.lax.fori_loop(0, n, body, 0)


def _edge_msgscatter(edge, exb, vrel):
    return pl.pallas_call(
        _msgscatter_body,
        grid=(_EPAD // _CHUNK,),
        in_specs=[
            pl.BlockSpec((2, _CHUNK), lambda i: (0, i),
                         memory_space=pltpu.SMEM),
            pl.BlockSpec((_CHUNK, _D), lambda i: (i, 0)),
            pl.BlockSpec((_N, _D), lambda i: (0, 0)),
        ],
        out_specs=pl.BlockSpec((_N, _D), lambda i: (0, 0)),
        out_shape=jax.ShapeDtypeStruct((_N, _D), jnp.float32),
    )(edge, exb, vrel)


def _denscatter_body(idx_ref, exb_ref, den_ref):
    @pl.when(pl.program_id(0) == 0)
    def _():
        den_ref[...] = jnp.zeros_like(den_ref)

    def body(e, carry):
        d = idx_ref[1, e]
        erow = exb_ref[pl.ds(e, 1), :]
        curd = den_ref[pl.ds(d, 1), :]
        den_ref[pl.ds(d, 1), :] = curd + erow
        return carry
    n = jnp.minimum(_CHUNK, _E - pl.program_id(0) * _CHUNK)
    jax.lax.fori_loop(0, n, body, 0)


def _edge_denscatter(edge, exb):
    return pl.pallas_call(
        _denscatter_body,
        grid=(_EPAD // _CHUNK,),
        in_specs=[
            pl.BlockSpec((2, _CHUNK), lambda i: (0, i),
                         memory_space=pltpu.SMEM),
            pl.BlockSpec((_CHUNK, _D), lambda i: (i, 0)),
        ],
        out_specs=pl.BlockSpec((_N, _D), lambda i: (0, 0)),
        out_shape=jax.ShapeDtypeStruct((_N, _D), jnp.float32),
    )(edge, exb)


def _post_body(num_ref, den_ref, x_ref, wa_ref, ba_ref, sk_ref, o_ref):
    outv = num_ref[...] / (den_ref[...] + 1e-16)
    g = 0.5 * outv * (1.0 + jax.lax.erf(outv * np.float32(1.0 / np.sqrt(2.0))))
    o = jnp.dot(g, wa_ref[...], preferred_element_type=jnp.float32) + ba_ref[...]
    sig = 1.0 / (1.0 + jnp.exp(-sk_ref[...]))
    o_ref[...] = jnp.maximum(sig * o + (1.0 - sig) * x_ref[...], 0.0)


def _post(num, den, x, wa, ba, sk):
    return pl.pallas_call(
        _post_body,
        grid=(_N // _ROWB,),
        in_specs=[
            pl.BlockSpec((_ROWB, _D), lambda i: (i, 0)),
            pl.BlockSpec((_ROWB, _D), lambda i: (i, 0)),
            pl.BlockSpec((_ROWB, _D), lambda i: (i, 0)),
            pl.BlockSpec((_D, _D), lambda i: (0, 0)),
            pl.BlockSpec((1, _D), lambda i: (0, 0)),
            pl.BlockSpec((1, _D), lambda i: (0, 0)),
        ],
        out_specs=pl.BlockSpec((_ROWB, _D), lambda i: (i, 0)),
        out_shape=jax.ShapeDtypeStruct((_N, _D), jnp.float32),
        compiler_params=pltpu.CompilerParams(
            dimension_semantics=("parallel",)),
    )(num, den, x, wa, ba.reshape(1, _D),
      jnp.full((1, _D), sk, jnp.float32))


def _block_diag(rel):
    out = jnp.zeros((_D, _D), jnp.float32)
    for h in range(_H):
        out = out.at[h * _DH:(h + 1) * _DH, h * _DH:(h + 1) * _DH].set(rel[h])
    return out


def _relation(x_src, x_dst, wk, bk_, wq, bq_, wv, bv_, arel, mrel, prel, edge):
    edge = jnp.concatenate(
        [edge, jnp.zeros((2, _EPAD - _E), jnp.int32)], axis=1)
    scale = prel[:, None, None] / np.float32(np.sqrt(_DH))
    bd_a = _block_diag(arel * scale)
    bd_m = _block_diag(mrel)
    krel = _proj(x_src, wk @ bd_a, bk_ @ bd_a)
    q = _proj(x_dst, wq, bq_)
    vrel = _proj(x_src, wv @ bd_m, bv_ @ bd_m)
    exb = _edge_alpha(edge, krel, q)
    num = _edge_msgscatter(edge, exb, vrel)
    den = _edge_denscatter(edge, exb)
    return num, den


def kernel(x_author, x_paper, edge_writes, edge_rev, Wk, Wq, Wv, Wa,
           bk, bq, bv, ba, skip, a_rel, m_rel, p_rel):
    xa, xp = x_author, x_paper
    for l in range(2):
        num_p, den_p = _relation(
            xa, xp, Wk[l, 0], bk[l, 0], Wq[l, 1], bq[l, 1], Wv[l, 0],
            bv[l, 0], a_rel[l, 0], m_rel[l, 0], p_rel[l, 0], edge_writes)
        num_a, den_a = _relation(
            xp, xa, Wk[l, 1], bk[l, 1], Wq[l, 0], bq[l, 0], Wv[l, 1],
            bv[l, 1], a_rel[l, 1], m_rel[l, 1], p_rel[l, 1], edge_rev)
        xa = _post(num_a, den_a, xa, Wa[l, 0], ba[l, 0], skip[l, 0])
        xp = _post(num_p, den_p, xp, Wa[l, 1], ba[l, 1], skip[l, 1])
    return xa, xp


# speedup vs baseline: 4.6422x; 4.6422x over previous
import functools
import numpy as np
import jax
import jax.numpy as jnp
from jax.experimental import pallas as pl
from jax.experimental.pallas import tpu as pltpu

_N = 50000
_D = 128
_H = 4
_DH = _D // _H
_E = 300000
_CHUNK = 1024
_EPAD = 300032
_ROWB = 2000

_HEAD_MASK = np.kron(np.eye(_H, dtype=np.float32),
                     np.ones((_DH, _DH), dtype=np.float32))


def _proj_body(x_ref, w_ref, b_ref, o_ref):
    o_ref[...] = (
        jnp.dot(x_ref[...], w_ref[...], preferred_element_type=jnp.float32)
        + b_ref[...]
    )


def _proj(x, w, b):
    return pl.pallas_call(
        _proj_body,
        grid=(_N // _ROWB,),
        in_specs=[
            pl.BlockSpec((_ROWB, _D), lambda i: (i, 0)),
            pl.BlockSpec((_D, _D), lambda i: (0, 0)),
            pl.BlockSpec((1, _D), lambda i: (0, 0)),
        ],
        out_specs=pl.BlockSpec((_ROWB, _D), lambda i: (i, 0)),
        out_shape=jax.ShapeDtypeStruct((_N, _D), jnp.float32),
        compiler_params=pltpu.CompilerParams(
            dimension_semantics=("parallel",)),
    )(x, w, b.reshape(1, _D))


def _alpha_body(idx_ref, krel_ref, q_ref, mask_ref, exb_ref):
    def body(i, carry):
        for j in range(8):
            e = i * 8 + j
            s = idx_ref[0, e]
            d = idx_ref[1, e]
            krow = krel_ref[pl.ds(s, 1), :]
            qrow = q_ref[pl.ds(d, 1), :]
            arow = jnp.dot(krow * qrow, mask_ref[...],
                           preferred_element_type=jnp.float32)
            exb_ref[pl.ds(e, 1), :] = jnp.exp(arow)
        return carry
    n = jnp.minimum(_CHUNK, _E - pl.program_id(0) * _CHUNK)
    jax.lax.fori_loop(0, n // 8, body, 0)


def _edge_alpha(edge, krel, q):
    mask = jnp.asarray(_HEAD_MASK)
    return pl.pallas_call(
        _alpha_body,
        grid=(_EPAD // _CHUNK,),
        in_specs=[
            pl.BlockSpec((2, _CHUNK), lambda i: (0, i),
                         memory_space=pltpu.SMEM),
            pl.BlockSpec((_N, _D), lambda i: (0, 0)),
            pl.BlockSpec((_N, _D), lambda i: (0, 0)),
            pl.BlockSpec((_D, _D), lambda i: (0, 0)),
        ],
        out_specs=pl.BlockSpec((_CHUNK, _D), lambda i: (i, 0)),
        out_shape=jax.ShapeDtypeStruct((_EPAD, _D), jnp.float32),
        compiler_params=pltpu.CompilerParams(
            dimension_semantics=("parallel",)),
    )(edge, krel, q, mask)


def _msgscatter_body(idx_ref, exb_ref, vrel_ref, num_ref):
    @pl.when(pl.program_id(0) == 0)
    def _():
        num_ref[...] = jnp.zeros_like(num_ref)

    def body(i, carry):
        for j in range(8):
            e = i * 8 + j
            s = idx_ref[0, e]
            d = idx_ref[1, e]
            vrow = vrel_ref[pl.ds(s, 1), :]
            exrow = exb_ref[pl.ds(e, 1), :]
            cur = num_ref[pl.ds(d, 1), :]
            num_ref[pl.ds(d, 1), :] = cur + exrow * vrow
        return carry
    n = jnp.minimum(_CHUNK, _E - pl.program_id(0) * _CHUNK)
    jax.lax.fori_loop(0, n // 8, body, 0)


def _edge_msgscatter(edge, exb, vrel):
    return pl.pallas_call(
        _msgscatter_body,
        grid=(_EPAD // _CHUNK,),
        in_specs=[
            pl.BlockSpec((2, _CHUNK), lambda i: (0, i),
                         memory_space=pltpu.SMEM),
            pl.BlockSpec((_CHUNK, _D), lambda i: (i, 0)),
            pl.BlockSpec((_N, _D), lambda i: (0, 0)),
        ],
        out_specs=pl.BlockSpec((_N, _D), lambda i: (0, 0)),
        out_shape=jax.ShapeDtypeStruct((_N, _D), jnp.float32),
    )(edge, exb, vrel)


def _denscatter_body(idx_ref, exb_ref, den_ref):
    @pl.when(pl.program_id(0) == 0)
    def _():
        den_ref[...] = jnp.zeros_like(den_ref)

    def body(i, carry):
        for j in range(8):
            e = i * 8 + j
            d = idx_ref[1, e]
            erow = exb_ref[pl.ds(e, 1), :]
            curd = den_ref[pl.ds(d, 1), :]
            den_ref[pl.ds(d, 1), :] = curd + erow
        return carry
    n = jnp.minimum(_CHUNK, _E - pl.program_id(0) * _CHUNK)
    jax.lax.fori_loop(0, n // 8, body, 0)


def _edge_denscatter(edge, exb):
    return pl.pallas_call(
        _denscatter_body,
        grid=(_EPAD // _CHUNK,),
        in_specs=[
            pl.BlockSpec((2, _CHUNK), lambda i: (0, i),
                         memory_space=pltpu.SMEM),
            pl.BlockSpec((_CHUNK, _D), lambda i: (i, 0)),
        ],
        out_specs=pl.BlockSpec((_N, _D), lambda i: (0, 0)),
        out_shape=jax.ShapeDtypeStruct((_N, _D), jnp.float32),
    )(edge, exb)


def _post_body(num_ref, den_ref, x_ref, wa_ref, ba_ref, sk_ref, o_ref):
    outv = num_ref[...] / (den_ref[...] + 1e-16)
    g = 0.5 * outv * (1.0 + jax.lax.erf(outv * np.float32(1.0 / np.sqrt(2.0))))
    o = jnp.dot(g, wa_ref[...], preferred_element_type=jnp.float32) + ba_ref[...]
    sig = 1.0 / (1.0 + jnp.exp(-sk_ref[...]))
    o_ref[...] = jnp.maximum(sig * o + (1.0 - sig) * x_ref[...], 0.0)


def _post(num, den, x, wa, ba, sk):
    return pl.pallas_call(
        _post_body,
        grid=(_N // _ROWB,),
        in_specs=[
            pl.BlockSpec((_ROWB, _D), lambda i: (i, 0)),
            pl.BlockSpec((_ROWB, _D), lambda i: (i, 0)),
            pl.BlockSpec((_ROWB, _D), lambda i: (i, 0)),
            pl.BlockSpec((_D, _D), lambda i: (0, 0)),
            pl.BlockSpec((1, _D), lambda i: (0, 0)),
            pl.BlockSpec((1, _D), lambda i: (0, 0)),
        ],
        out_specs=pl.BlockSpec((_ROWB, _D), lambda i: (i, 0)),
        out_shape=jax.ShapeDtypeStruct((_N, _D), jnp.float32),
        compiler_params=pltpu.CompilerParams(
            dimension_semantics=("parallel",)),
    )(num, den, x, wa, ba.reshape(1, _D),
      jnp.full((1, _D), sk, jnp.float32))


def _block_diag(rel):
    out = jnp.zeros((_D, _D), jnp.float32)
    for h in range(_H):
        out = out.at[h * _DH:(h + 1) * _DH, h * _DH:(h + 1) * _DH].set(rel[h])
    return out


def _relation(x_src, x_dst, wk, bk_, wq, bq_, wv, bv_, arel, mrel, prel, edge):
    edge = jnp.concatenate(
        [edge, jnp.zeros((2, _EPAD - _E), jnp.int32)], axis=1)
    scale = prel[:, None, None] / np.float32(np.sqrt(_DH))
    bd_a = _block_diag(arel * scale)
    bd_m = _block_diag(mrel)
    krel = _proj(x_src, wk @ bd_a, bk_ @ bd_a)
    q = _proj(x_dst, wq, bq_)
    vrel = _proj(x_src, wv @ bd_m, bv_ @ bd_m)
    exb = _edge_alpha(edge, krel, q)
    num = _edge_msgscatter(edge, exb, vrel)
    den = _edge_denscatter(edge, exb)
    return num, den


def kernel(x_author, x_paper, edge_writes, edge_rev, Wk, Wq, Wv, Wa,
           bk, bq, bv, ba, skip, a_rel, m_rel, p_rel):
    xa, xp = x_author, x_paper
    for l in range(2):
        num_p, den_p = _relation(
            xa, xp, Wk[l, 0], bk[l, 0], Wq[l, 1], bq[l, 1], Wv[l, 0],
            bv[l, 0], a_rel[l, 0], m_rel[l, 0], p_rel[l, 0], edge_writes)
        num_a, den_a = _relation(
            xp, xa, Wk[l, 1], bk[l, 1], Wq[l, 0], bq[l, 0], Wv[l, 1],
            bv[l, 1], a_rel[l, 1], m_rel[l, 1], p_rel[l, 1], edge_rev)
        xa = _post(num_a, den_a, xa, Wa[l, 0], ba[l, 0], skip[l, 0])
        xp = _post(num_p, den_p, xp, Wa[l, 1], ba[l, 1], skip[l, 1])
    return xa, xp


# speedup vs baseline: 6.1319x; 1.3209x over previous
import functools
import numpy as np
import jax
import jax.numpy as jnp
from jax.experimental import pallas as pl
from jax.experimental.pallas import tpu as pltpu

_N = 50000
_D = 128
_H = 4
_DH = _D // _H
_E = 300000
_CHUNK = 1024
_EPAD = 300032
_ROWB = 2000

_HEAD_MASK = np.kron(np.eye(_H, dtype=np.float32),
                     np.ones((_DH, _DH), dtype=np.float32))


def _proj_body(x_ref, w_ref, b_ref, o_ref):
    o_ref[...] = (
        jnp.dot(x_ref[...], w_ref[...], preferred_element_type=jnp.float32)
        + b_ref[...]
    )


def _proj(x, w, b):
    return pl.pallas_call(
        _proj_body,
        grid=(_N // _ROWB,),
        in_specs=[
            pl.BlockSpec((_ROWB, _D), lambda i: (i, 0)),
            pl.BlockSpec((_D, _D), lambda i: (0, 0)),
            pl.BlockSpec((1, _D), lambda i: (0, 0)),
        ],
        out_specs=pl.BlockSpec((_ROWB, _D), lambda i: (i, 0)),
        out_shape=jax.ShapeDtypeStruct((_N, _D), jnp.float32),
        compiler_params=pltpu.CompilerParams(
            dimension_semantics=("parallel",)),
    )(x, w, b.reshape(1, _D))


def _alpha_body(idx_ref, krel_ref, q_ref, mask_ref, exb_ref):
    def body(i, carry):
        for j in range(16):
            e = i * 16 + j
            s = idx_ref[0, e]
            d = idx_ref[1, e]
            krow = krel_ref[pl.ds(s, 1), :]
            qrow = q_ref[pl.ds(d, 1), :]
            arow = jnp.dot(krow * qrow, mask_ref[...],
                           preferred_element_type=jnp.float32)
            exb_ref[pl.ds(e, 1), :] = jnp.exp(arow)
        return carry
    n = jnp.minimum(_CHUNK, _E - pl.program_id(0) * _CHUNK)
    jax.lax.fori_loop(0, n // 16, body, 0)


def _edge_alpha(edge, krel, q):
    mask = jnp.asarray(_HEAD_MASK)
    return pl.pallas_call(
        _alpha_body,
        grid=(_EPAD // _CHUNK,),
        in_specs=[
            pl.BlockSpec((2, _CHUNK), lambda i: (0, i),
                         memory_space=pltpu.SMEM),
            pl.BlockSpec((_N, _D), lambda i: (0, 0)),
            pl.BlockSpec((_N, _D), lambda i: (0, 0)),
            pl.BlockSpec((_D, _D), lambda i: (0, 0)),
        ],
        out_specs=pl.BlockSpec((_CHUNK, _D), lambda i: (i, 0)),
        out_shape=jax.ShapeDtypeStruct((_EPAD, _D), jnp.float32),
        compiler_params=pltpu.CompilerParams(
            dimension_semantics=("parallel",)),
    )(edge, krel, q, mask)


def _msgscatter_body(idx_ref, exb_ref, vrel_ref, num_ref):
    @pl.when(pl.program_id(0) == 0)
    def _():
        num_ref[...] = jnp.zeros_like(num_ref)

    def body(i, carry):
        for j in range(16):
            e = i * 16 + j
            s = idx_ref[0, e]
            d = idx_ref[1, e]
            vrow = vrel_ref[pl.ds(s, 1), :]
            exrow = exb_ref[pl.ds(e, 1), :]
            cur = num_ref[pl.ds(d, 1), :]
            num_ref[pl.ds(d, 1), :] = cur + exrow * vrow
        return carry
    n = jnp.minimum(_CHUNK, _E - pl.program_id(0) * _CHUNK)
    jax.lax.fori_loop(0, n // 16, body, 0)


def _edge_msgscatter(edge, exb, vrel):
    return pl.pallas_call(
        _msgscatter_body,
        grid=(_EPAD // _CHUNK,),
        in_specs=[
            pl.BlockSpec((2, _CHUNK), lambda i: (0, i),
                         memory_space=pltpu.SMEM),
            pl.BlockSpec((_CHUNK, _D), lambda i: (i, 0)),
            pl.BlockSpec((_N, _D), lambda i: (0, 0)),
        ],
        out_specs=pl.BlockSpec((_N, _D), lambda i: (0, 0)),
        out_shape=jax.ShapeDtypeStruct((_N, _D), jnp.float32),
    )(edge, exb, vrel)


def _denscatter_body(idx_ref, exb_ref, den_ref):
    @pl.when(pl.program_id(0) == 0)
    def _():
        den_ref[...] = jnp.zeros_like(den_ref)

    def body(i, carry):
        for j in range(16):
            e = i * 16 + j
            d = idx_ref[1, e]
            erow = exb_ref[pl.ds(e, 1), :]
            curd = den_ref[pl.ds(d, 1), :]
            den_ref[pl.ds(d, 1), :] = curd + erow
        return carry
    n = jnp.minimum(_CHUNK, _E - pl.program_id(0) * _CHUNK)
    jax.lax.fori_loop(0, n // 16, body, 0)


def _edge_denscatter(edge, exb):
    return pl.pallas_call(
        _denscatter_body,
        grid=(_EPAD // _CHUNK,),
        in_specs=[
            pl.BlockSpec((2, _CHUNK), lambda i: (0, i),
                         memory_space=pltpu.SMEM),
            pl.BlockSpec((_CHUNK, _D), lambda i: (i, 0)),
        ],
        out_specs=pl.BlockSpec((_N, _D), lambda i: (0, 0)),
        out_shape=jax.ShapeDtypeStruct((_N, _D), jnp.float32),
    )(edge, exb)


def _post_body(num_ref, den_ref, x_ref, wa_ref, ba_ref, sk_ref, o_ref):
    outv = num_ref[...] / (den_ref[...] + 1e-16)
    g = 0.5 * outv * (1.0 + jax.lax.erf(outv * np.float32(1.0 / np.sqrt(2.0))))
    o = jnp.dot(g, wa_ref[...], preferred_element_type=jnp.float32) + ba_ref[...]
    sig = 1.0 / (1.0 + jnp.exp(-sk_ref[...]))
    o_ref[...] = jnp.maximum(sig * o + (1.0 - sig) * x_ref[...], 0.0)


def _post(num, den, x, wa, ba, sk):
    return pl.pallas_call(
        _post_body,
        grid=(_N // _ROWB,),
        in_specs=[
            pl.BlockSpec((_ROWB, _D), lambda i: (i, 0)),
            pl.BlockSpec((_ROWB, _D), lambda i: (i, 0)),
            pl.BlockSpec((_ROWB, _D), lambda i: (i, 0)),
            pl.BlockSpec((_D, _D), lambda i: (0, 0)),
            pl.BlockSpec((1, _D), lambda i: (0, 0)),
            pl.BlockSpec((1, _D), lambda i: (0, 0)),
        ],
        out_specs=pl.BlockSpec((_ROWB, _D), lambda i: (i, 0)),
        out_shape=jax.ShapeDtypeStruct((_N, _D), jnp.float32),
        compiler_params=pltpu.CompilerParams(
            dimension_semantics=("parallel",)),
    )(num, den, x, wa, ba.reshape(1, _D),
      jnp.full((1, _D), sk, jnp.float32))


def _block_diag(rel):
    out = jnp.zeros((_D, _D), jnp.float32)
    for h in range(_H):
        out = out.at[h * _DH:(h + 1) * _DH, h * _DH:(h + 1) * _DH].set(rel[h])
    return out


def _relation(x_src, x_dst, wk, bk_, wq, bq_, wv, bv_, arel, mrel, prel, edge):
    edge = jnp.concatenate(
        [edge, jnp.zeros((2, _EPAD - _E), jnp.int32)], axis=1)
    scale = prel[:, None, None] / np.float32(np.sqrt(_DH))
    bd_a = _block_diag(arel * scale)
    bd_m = _block_diag(mrel)
    krel = _proj(x_src, wk @ bd_a, bk_ @ bd_a)
    q = _proj(x_dst, wq, bq_)
    vrel = _proj(x_src, wv @ bd_m, bv_ @ bd_m)
    exb = _edge_alpha(edge, krel, q)
    num = _edge_msgscatter(edge, exb, vrel)
    den = _edge_denscatter(edge, exb)
    return num, den


def kernel(x_author, x_paper, edge_writes, edge_rev, Wk, Wq, Wv, Wa,
           bk, bq, bv, ba, skip, a_rel, m_rel, p_rel):
    xa, xp = x_author, x_paper
    for l in range(2):
        num_p, den_p = _relation(
            xa, xp, Wk[l, 0], bk[l, 0], Wq[l, 1], bq[l, 1], Wv[l, 0],
            bv[l, 0], a_rel[l, 0], m_rel[l, 0], p_rel[l, 0], edge_writes)
        num_a, den_a = _relation(
            xp, xa, Wk[l, 1], bk[l, 1], Wq[l, 0], bq[l, 0], Wv[l, 1],
            bv[l, 1], a_rel[l, 1], m_rel[l, 1], p_rel[l, 1], edge_rev)
        xa = _post(num_a, den_a, xa, Wa[l, 0], ba[l, 0], skip[l, 0])
        xp = _post(num_p, den_p, xp, Wa[l, 1], ba[l, 1], skip[l, 1])
    return xa, xp


# speedup vs baseline: 7.2951x; 1.1897x over previous
import functools
import numpy as np
import jax
import jax.numpy as jnp
from jax.experimental import pallas as pl
from jax.experimental.pallas import tpu as pltpu

_N = 50000
_D = 128
_H = 4
_DH = _D // _H
_E = 300000
_CHUNK = 1024
_EPAD = 300032
_ROWB = 2000

_HEAD_MASK = np.kron(np.eye(_H, dtype=np.float32),
                     np.ones((_DH, _DH), dtype=np.float32))


def _proj_body(x_ref, w_ref, b_ref, o_ref):
    o_ref[...] = (
        jnp.dot(x_ref[...], w_ref[...], preferred_element_type=jnp.float32)
        + b_ref[...]
    )


def _proj(x, w, b):
    return pl.pallas_call(
        _proj_body,
        grid=(_N // _ROWB,),
        in_specs=[
            pl.BlockSpec((_ROWB, _D), lambda i: (i, 0)),
            pl.BlockSpec((_D, _D), lambda i: (0, 0)),
            pl.BlockSpec((1, _D), lambda i: (0, 0)),
        ],
        out_specs=pl.BlockSpec((_ROWB, _D), lambda i: (i, 0)),
        out_shape=jax.ShapeDtypeStruct((_N, _D), jnp.float32),
        compiler_params=pltpu.CompilerParams(
            dimension_semantics=("parallel",)),
    )(x, w, b.reshape(1, _D))


def _alpha_body(idx_ref, krel_ref, q_ref, mask_ref, exb_ref):
    def body(i, carry):
        for j in range(32):
            e = i * 32 + j
            s = idx_ref[0, e]
            d = idx_ref[1, e]
            krow = krel_ref[pl.ds(s, 1), :]
            qrow = q_ref[pl.ds(d, 1), :]
            arow = jnp.dot(krow * qrow, mask_ref[...],
                           preferred_element_type=jnp.float32)
            exb_ref[pl.ds(e, 1), :] = jnp.exp(arow)
        return carry
    n = jnp.minimum(_CHUNK, _E - pl.program_id(0) * _CHUNK)
    jax.lax.fori_loop(0, n // 32, body, 0)


def _edge_alpha(edge, krel, q):
    mask = jnp.asarray(_HEAD_MASK)
    return pl.pallas_call(
        _alpha_body,
        grid=(_EPAD // _CHUNK,),
        in_specs=[
            pl.BlockSpec((2, _CHUNK), lambda i: (0, i),
                         memory_space=pltpu.SMEM),
            pl.BlockSpec((_N, _D), lambda i: (0, 0)),
            pl.BlockSpec((_N, _D), lambda i: (0, 0)),
            pl.BlockSpec((_D, _D), lambda i: (0, 0)),
        ],
        out_specs=pl.BlockSpec((_CHUNK, _D), lambda i: (i, 0)),
        out_shape=jax.ShapeDtypeStruct((_EPAD, _D), jnp.float32),
        compiler_params=pltpu.CompilerParams(
            dimension_semantics=("parallel",)),
    )(edge, krel, q, mask)


def _msgscatter_body(idx_ref, exb_ref, vrel_ref, num_ref):
    @pl.when(pl.program_id(0) == 0)
    def _():
        num_ref[...] = jnp.zeros_like(num_ref)

    def body(i, carry):
        for j in range(32):
            e = i * 32 + j
            s = idx_ref[0, e]
            d = idx_ref[1, e]
            vrow = vrel_ref[pl.ds(s, 1), :]
            exrow = exb_ref[pl.ds(e, 1), :]
            cur = num_ref[pl.ds(d, 1), :]
            num_ref[pl.ds(d, 1), :] = cur + exrow * vrow
        return carry
    n = jnp.minimum(_CHUNK, _E - pl.program_id(0) * _CHUNK)
    jax.lax.fori_loop(0, n // 32, body, 0)


def _edge_msgscatter(edge, exb, vrel):
    return pl.pallas_call(
        _msgscatter_body,
        grid=(_EPAD // _CHUNK,),
        in_specs=[
            pl.BlockSpec((2, _CHUNK), lambda i: (0, i),
                         memory_space=pltpu.SMEM),
            pl.BlockSpec((_CHUNK, _D), lambda i: (i, 0)),
            pl.BlockSpec((_N, _D), lambda i: (0, 0)),
        ],
        out_specs=pl.BlockSpec((_N, _D), lambda i: (0, 0)),
        out_shape=jax.ShapeDtypeStruct((_N, _D), jnp.float32),
    )(edge, exb, vrel)


def _denscatter_body(idx_ref, exb_ref, den_ref):
    @pl.when(pl.program_id(0) == 0)
    def _():
        den_ref[...] = jnp.zeros_like(den_ref)

    def body(i, carry):
        for j in range(32):
            e = i * 32 + j
            d = idx_ref[1, e]
            erow = exb_ref[pl.ds(e, 1), :]
            curd = den_ref[pl.ds(d, 1), :]
            den_ref[pl.ds(d, 1), :] = curd + erow
        return carry
    n = jnp.minimum(_CHUNK, _E - pl.program_id(0) * _CHUNK)
    jax.lax.fori_loop(0, n // 32, body, 0)


def _edge_denscatter(edge, exb):
    return pl.pallas_call(
        _denscatter_body,
        grid=(_EPAD // _CHUNK,),
        in_specs=[
            pl.BlockSpec((2, _CHUNK), lambda i: (0, i),
                         memory_space=pltpu.SMEM),
            pl.BlockSpec((_CHUNK, _D), lambda i: (i, 0)),
        ],
        out_specs=pl.BlockSpec((_N, _D), lambda i: (0, 0)),
        out_shape=jax.ShapeDtypeStruct((_N, _D), jnp.float32),
    )(edge, exb)


def _post_body(num_ref, den_ref, x_ref, wa_ref, ba_ref, sk_ref, o_ref):
    outv = num_ref[...] / (den_ref[...] + 1e-16)
    g = 0.5 * outv * (1.0 + jax.lax.erf(outv * np.float32(1.0 / np.sqrt(2.0))))
    o = jnp.dot(g, wa_ref[...], preferred_element_type=jnp.float32) + ba_ref[...]
    sig = 1.0 / (1.0 + jnp.exp(-sk_ref[...]))
    o_ref[...] = jnp.maximum(sig * o + (1.0 - sig) * x_ref[...], 0.0)


def _post(num, den, x, wa, ba, sk):
    return pl.pallas_call(
        _post_body,
        grid=(_N // _ROWB,),
        in_specs=[
            pl.BlockSpec((_ROWB, _D), lambda i: (i, 0)),
            pl.BlockSpec((_ROWB, _D), lambda i: (i, 0)),
            pl.BlockSpec((_ROWB, _D), lambda i: (i, 0)),
            pl.BlockSpec((_D, _D), lambda i: (0, 0)),
            pl.BlockSpec((1, _D), lambda i: (0, 0)),
            pl.BlockSpec((1, _D), lambda i: (0, 0)),
        ],
        out_specs=pl.BlockSpec((_ROWB, _D), lambda i: (i, 0)),
        out_shape=jax.ShapeDtypeStruct((_N, _D), jnp.float32),
        compiler_params=pltpu.CompilerParams(
            dimension_semantics=("parallel",)),
    )(num, den, x, wa, ba.reshape(1, _D),
      jnp.full((1, _D), sk, jnp.float32))


def _block_diag(rel):
    out = jnp.zeros((_D, _D), jnp.float32)
    for h in range(_H):
        out = out.at[h * _DH:(h + 1) * _DH, h * _DH:(h + 1) * _DH].set(rel[h])
    return out


def _relation(x_src, x_dst, wk, bk_, wq, bq_, wv, bv_, arel, mrel, prel, edge):
    edge = jnp.concatenate(
        [edge, jnp.zeros((2, _EPAD - _E), jnp.int32)], axis=1)
    scale = prel[:, None, None] / np.float32(np.sqrt(_DH))
    bd_a = _block_diag(arel * scale)
    bd_m = _block_diag(mrel)
    krel = _proj(x_src, wk @ bd_a, bk_ @ bd_a)
    q = _proj(x_dst, wq, bq_)
    vrel = _proj(x_src, wv @ bd_m, bv_ @ bd_m)
    exb = _edge_alpha(edge, krel, q)
    num = _edge_msgscatter(edge, exb, vrel)
    den = _edge_denscatter(edge, exb)
    return num, den


def kernel(x_author, x_paper, edge_writes, edge_rev, Wk, Wq, Wv, Wa,
           bk, bq, bv, ba, skip, a_rel, m_rel, p_rel):
    xa, xp = x_author, x_paper
    for l in range(2):
        num_p, den_p = _relation(
            xa, xp, Wk[l, 0], bk[l, 0], Wq[l, 1], bq[l, 1], Wv[l, 0],
            bv[l, 0], a_rel[l, 0], m_rel[l, 0], p_rel[l, 0], edge_writes)
        num_a, den_a = _relation(
            xp, xa, Wk[l, 1], bk[l, 1], Wq[l, 0], bq[l, 0], Wv[l, 1],
            bv[l, 1], a_rel[l, 1], m_rel[l, 1], p_rel[l, 1], edge_rev)
        xa = _post(num_a, den_a, xa, Wa[l, 0], ba[l, 0], skip[l, 0])
        xp = _post(num_p, den_p, xp, Wa[l, 1], ba[l, 1], skip[l, 1])
    return xa, xp
